# Initial kernel scaffold; baseline (speedup 1.0000x reference)
import jax
import jax.numpy as jnp
from jax.experimental import pallas as pl


def kernel(input, Wp, bp, dwk, dwb, gamma, beta, Wo, bo, Wm, bm, Wout, bout):
    raise NotImplementedError("write your pallas kernel here")


# speedup vs baseline: 97.0178x; 97.0178x over previous
import functools

import jax
import jax.numpy as jnp
import numpy as np
from jax import lax
from jax.experimental import pallas as pl
from jax.experimental.pallas import tpu as pltpu
from jax.experimental.pallas import tpu_sc as plsc

N, H, W, C = 2, 224, 224, 96
G, K = 6, 3
P = K * K
GC = C // G
GP = G * P
HW = H * W

RB = 28
NHB = H // RB

NROWS = N * HW * G
NW = 32
RPT = NROWS // NW
R = 192
NCHUNK = RPT // R

_SQ2PI = float(np.sqrt(2.0 / np.pi))


def _proj_body(inp, x1b, wp, bp, woy, boy, wox, box, wm, bm,
               segA, segB, dyv, dxv, gcol, xo, io, mo):
    n = pl.program_id(0)
    hb = pl.program_id(1)

    x1f = x1b[0].reshape(RB * W, C)
    offy = jnp.dot(x1f, woy[...], preferred_element_type=jnp.float32) + boy[0]
    offx = jnp.dot(x1f, wox[...], preferred_element_type=jnp.float32) + box[0]
    logits = jnp.dot(x1f, wm[...], preferred_element_type=jnp.float32) + bm[0]
    e = jnp.exp(logits)
    den = jnp.dot(jnp.dot(e, segA[...], preferred_element_type=jnp.float32),
                  segB[...], preferred_element_type=jnp.float32)
    m = e / den

    xpf = jnp.dot(inp[0].reshape(RB * W, C), wp[...],
                  preferred_element_type=jnp.float32) + bp[0]
    xo[0] = xpf.reshape(RB, W, C)

    wio = lax.broadcasted_iota(jnp.int32, (RB, W, GP), 1).astype(jnp.float32)
    hio = (lax.broadcasted_iota(jnp.int32, (RB, W, GP), 0)
           + hb * RB).astype(jnp.float32)
    rawy = wio + 1.5 + dyv[0] + offy.reshape(RB, W, GP)
    rawx = hio + 1.5 + dxv[0] + offx.reshape(RB, W, GP)
    ty = jnp.clip(jnp.floor(rawy).astype(jnp.int32), 0, H + 1)
    tx = jnp.clip(jnp.floor(rawx).astype(jnp.int32), 0, W + 1)
    valid = (ty >= 1) & (ty <= H) & (tx >= 1) & (tx <= W)
    iyu = jnp.clip(ty - 1, 0, H - 1)
    ixu = jnp.clip(tx - 1, 0, W - 1)
    row = (n * HW + iyu * W + ixu) * G + gcol[0]
    io[0] = row
    mo[0] = m.reshape(RB, W, GP) * valid.astype(jnp.float32)


def _gather_body(xt_hbm, idx_hbm, mv_hbm, out_hbm, idx_v, mv_v, rows_v,
                 acc_v, sem):
    wid = lax.axis_index("s") * 2 + lax.axis_index("c")
    base = wid * RPT

    def chunk(ci, carry):
        r0 = base + ci * R
        pltpu.sync_copy(idx_hbm.at[pl.ds(r0 * P, R * P)], idx_v)
        pltpu.sync_copy(mv_hbm.at[pl.ds(r0 * P, R * P)],
                        mv_v.at[pl.ds(0, R * P)])
        pltpu.async_copy(xt_hbm.at[idx_v], rows_v, sem).wait()

        def rowfn(r, c2):
            b = r * P
            mv9 = mv_v[pl.ds(b, 16)]
            a = rows_v[b] * mv9[0]
            for p in range(1, P):
                a = a + rows_v[b + p] * mv9[p]
            acc_v[r] = a
            return c2

        lax.fori_loop(0, R, rowfn, 0)
        pltpu.sync_copy(acc_v, out_hbm.at[pl.ds(r0, R)])
        return carry

    lax.fori_loop(0, NCHUNK, chunk, 0)


def _out_body(y, wout, bout, o):
    o[...] = jnp.dot(y[...], wout[...],
                     preferred_element_type=jnp.float32) + bout[0]


def _stage_proj(input, Wp, bp, dwk, dwb, gamma, beta, Wo, bo, Wm, bm):
    f32 = jnp.float32

    dw = lax.conv_general_dilated(
        input, dwk, (1, 1), 'SAME',
        dimension_numbers=('NHWC', 'HWIO', 'NHWC'),
        feature_group_count=C) + dwb
    mu = jnp.mean(dw, -1, keepdims=True)
    var = jnp.mean((dw - mu) ** 2, -1, keepdims=True)
    x1 = (dw - mu) / jnp.sqrt(var + 1e-6) * gamma + beta
    x1 = 0.5 * x1 * (1 + jnp.tanh(_SQ2PI * (x1 + 0.044715 * x1 ** 3)))

    woy, wox = Wo[:, 0::2], Wo[:, 1::2]
    boy, box = bo[0::2].reshape(1, GP), bo[1::2].reshape(1, GP)

    segA = jnp.asarray(np.kron(np.eye(G, dtype=np.float32),
                               np.ones((P, 1), np.float32)))
    segB = jnp.asarray(segA.T)
    pts = np.array([-1.0, 0.0, 1.0], np.float32)
    dyv = jnp.asarray(np.tile(np.repeat(pts, K), G).reshape(1, GP))
    dxv = jnp.asarray(np.tile(pts, K * G).reshape(1, GP))
    gcol = jnp.asarray(np.repeat(np.arange(G, dtype=np.int32), P).reshape(1, GP))

    fixed = lambda *shape: pl.BlockSpec(shape, lambda n, h: (0,) * len(shape))
    x_proj, idx, mval = pl.pallas_call(
        _proj_body,
        grid=(N, NHB),
        in_specs=[
            pl.BlockSpec((1, RB, W, C), lambda n, h: (n, h, 0, 0)),
            pl.BlockSpec((1, RB, W, C), lambda n, h: (n, h, 0, 0)),
            fixed(C, C), fixed(1, C),
            fixed(C, GP), fixed(1, GP), fixed(C, GP), fixed(1, GP),
            fixed(C, GP), fixed(1, GP), fixed(GP, G), fixed(G, GP),
            fixed(1, GP), fixed(1, GP), fixed(1, GP),
        ],
        out_specs=[
            pl.BlockSpec((1, RB, W, C), lambda n, h: (n, h, 0, 0)),
            pl.BlockSpec((1, RB, W, GP), lambda n, h: (n, h, 0, 0)),
            pl.BlockSpec((1, RB, W, GP), lambda n, h: (n, h, 0, 0)),
        ],
        out_shape=[
            jax.ShapeDtypeStruct((N, H, W, C), f32),
            jax.ShapeDtypeStruct((N, H, W, GP), jnp.int32),
            jax.ShapeDtypeStruct((N, H, W, GP), f32),
        ],
    )(input, x1, Wp, bp.reshape(1, C), woy, boy, wox, box,
      Wm, bm.reshape(1, GP), segA, segB, dyv, dxv, gcol)
    return x_proj, idx, mval


def _stage_gather(x_proj, idx, mval):
    f32 = jnp.float32
    gcall = pl.kernel(
        _gather_body,
        out_type=jax.ShapeDtypeStruct((NROWS, GC), f32),
        mesh=plsc.VectorSubcoreMesh(core_axis_name="c", subcore_axis_name="s",
                                    num_cores=2, num_subcores=16),
        scratch_types=[
            pltpu.VMEM((R * P,), jnp.int32),
            pltpu.VMEM((R * P + 16,), f32),
            pltpu.VMEM((R * P, GC), f32),
            pltpu.VMEM((R, GC), f32),
            pltpu.SemaphoreType.DMA,
        ],
        compiler_params=pltpu.CompilerParams(use_tc_tiling_on_sc=False),
    )
    return gcall(x_proj.reshape(NROWS, GC), idx.reshape(NROWS * P),
                 mval.reshape(NROWS * P))


def _stage_out(y, Wout, bout):
    f32 = jnp.float32
    MB = 2048
    out = pl.pallas_call(
        _out_body,
        grid=(N * HW // MB,),
        in_specs=[
            pl.BlockSpec((MB, C), lambda i: (i, 0)),
            pl.BlockSpec((C, C), lambda i: (0, 0)),
            pl.BlockSpec((1, C), lambda i: (0, 0)),
        ],
        out_specs=pl.BlockSpec((MB, C), lambda i: (i, 0)),
        out_shape=jax.ShapeDtypeStruct((N * HW, C), f32),
    )(y.reshape(N * HW, C), Wout, bout.reshape(1, C))
    return out.reshape(N, H, W, C)


def kernel(input, Wp, bp, dwk, dwb, gamma, beta, Wo, bo, Wm, bm, Wout, bout):
    x_proj, idx, mval = _stage_proj(input, Wp, bp, dwk, dwb, gamma, beta,
                                    Wo, bo, Wm, bm)
    y = _stage_gather(x_proj, idx, mval)
    return _stage_out(y, Wout, bout)
